# baseline (speedup 1.0000x reference)
import functools

import jax
import jax.numpy as jnp
from jax import lax
from jax.experimental import pallas as pl
from jax.experimental.pallas import tpu as pltpu
from jax.experimental.pallas import tpu_sc as plsc

NC, NS = 2, 16
NW = NC * NS
D = 32
L = 16

B = 16384
H = 200

BSUB = 64
HSUB = 10
ROWS = BSUB * HSUB
GBLK = 128
NG = (ROWS + GBLK - 1) // GBLK


def _sc_gather_t(table, idxT):
    b_per_w = B // NW
    n_bsub = b_per_w // BSUB
    n_hsub = H // HSUB

    mesh = plsc.VectorSubcoreMesh(
        core_axis_name="c", subcore_axis_name="s",
        num_cores=NC, num_subcores=NS)

    @functools.partial(
        pl.kernel,
        out_type=jax.ShapeDtypeStruct((H, D // 8, B // 128, 8, 128), jnp.float32),
        mesh=mesh,
        scratch_types=[
            pltpu.VMEM((2, HSUB, BSUB), jnp.int32),
            pltpu.VMEM((2, NG, GBLK), jnp.int32),
            pltpu.VMEM((2, NG * GBLK, D), jnp.float32),
            pltpu.VMEM((2, HSUB, D // 8, 1, 8, BSUB + 1), jnp.float32),
            pltpu.SemaphoreType.DMA((2,)),
            pltpu.SemaphoreType.DMA((2,)),
            pltpu.SemaphoreType.DMA((2,)),
        ],
        compiler_params=pltpu.CompilerParams(
            use_tc_tiling_on_sc=False, needs_layout_passes=False),
    )
    def k(table_hbm, idx_hbm, out_hbm, idx_v, gl_v, rows_v, tr_v,
          sem_i, sem_g, sem_o):
        wid = lax.axis_index("s") * NC + lax.axis_index("c")
        b_base = wid * b_per_w
        iota = lax.broadcasted_iota(jnp.int32, (L,), 0)

        def idx_copy(s, buf):
            h0 = (s % n_hsub) * HSUB
            b0 = b_base + (s // n_hsub) * BSUB
            return pltpu.make_async_copy(
                idx_hbm.at[pl.ds(h0, HSUB), pl.ds(b0, BSUB)],
                idx_v.at[buf], sem_i.at[buf])

        def out_copy(s, buf):
            h0 = (s % n_hsub) * HSUB
            b0 = b_base + (s // n_hsub) * BSUB
            return pltpu.make_async_copy(
                tr_v.at[buf, :, :, :, :, pl.ds(0, BSUB)],
                out_hbm.at[pl.ds(h0, HSUB), :, pl.ds(b0 // 128, 1), :,
                           pl.ds(b0 % 128, BSUB)],
                sem_o.at[buf])

        n_sub = n_bsub * n_hsub

        for buf in range(2):
            idx_copy(buf, buf).start()

        def step2(i2, carry):
            for buf in range(2):
                s = i2 * 2 + buf
                idx_copy(s, buf).wait()
                for hh in range(HSUB):
                    for half in range(BSUB // L):
                        v = idx_v[buf, hh, pl.ds(half * L, L)]
                        q = hh * (BSUB // L) + half
                        gl_v[buf, q // 8, pl.ds((q % 8) * L, L)] = v
                vpad = idx_v[buf, 0, pl.ds(0, L)]
                for q in range(ROWS // L, NG * GBLK // L):
                    gl_v[buf, q // 8, pl.ds((q % 8) * L, L)] = vpad
                gathers = [
                    pltpu.async_copy(
                        table_hbm.at[gl_v.at[buf, g]],
                        rows_v.at[buf, pl.ds(g * GBLK, GBLK), :],
                        sem_g.at[buf])
                    for g in range(NG)
                ]
                for gcp in gathers:
                    gcp.wait()
                @pl.when(s >= 2)
                def _():
                    out_copy(s - 2, buf).wait()

                zeros16 = jnp.zeros((L,), jnp.int32)
                i8a = iota // 8
                i8b = i8a + 2
                m8 = lax.rem(iota, jnp.full((L,), 8, jnp.int32))

                @plsc.parallel_loop(0, ROWS, unroll=16)
                def _(r):
                    hh = r // BSUB
                    bb = r % BSUB
                    bsplat = jnp.full((L,), bb, jnp.int32)
                    v0 = rows_v[buf, r, pl.ds(0, L)]
                    v1 = rows_v[buf, r, pl.ds(L, L)]
                    sub = tr_v.at[buf, hh]
                    plsc.store_scatter(sub, [i8a, zeros16, m8, bsplat], v0)
                    plsc.store_scatter(sub, [i8b, zeros16, m8, bsplat], v1)
                out_copy(s, buf).start()
                @pl.when(s + 2 < n_sub)
                def _():
                    idx_copy(s + 2, buf).start()
            return carry

        lax.fori_loop(0, n_sub // 2, step2, 0)
        for buf in range(2):
            out_copy(n_sub - 2 + buf, buf).wait()

    return k(table, idxT)


def kernel(idx, W_train, W_frozen):
    table = jnp.concatenate([W_train, W_frozen], axis=0)
    out5 = _sc_gather_t(table, idx.T)
    out = jnp.transpose(out5, (2, 4, 0, 1, 3))
    return out.reshape(B, H, D)


# speedup vs baseline: 1.0799x; 1.0799x over previous
import functools

import jax
import jax.numpy as jnp
from jax import lax
from jax.experimental import pallas as pl
from jax.experimental.pallas import tpu as pltpu
from jax.experimental.pallas import tpu_sc as plsc

NC, NS = 2, 16
NW = NC * NS
D = 32
L = 16

B = 16384
H = 200

BSUB = 64
HSUB = 10
ROWS = BSUB * HSUB
GBLK = 128
NG = (ROWS + GBLK - 1) // GBLK


def _sc_gather_t(table, idxT):
    b_per_w = B // NW
    n_bsub = b_per_w // BSUB
    n_hsub = H // HSUB

    mesh = plsc.VectorSubcoreMesh(
        core_axis_name="c", subcore_axis_name="s",
        num_cores=NC, num_subcores=NS)

    @functools.partial(
        pl.kernel,
        out_type=jax.ShapeDtypeStruct((H, D // 8, B // 128, 8, 128), jnp.float32),
        mesh=mesh,
        scratch_types=[
            pltpu.VMEM((2, HSUB, BSUB), jnp.int32),
            pltpu.VMEM((2, NG, GBLK), jnp.int32),
            pltpu.VMEM((2, NG * GBLK, D), jnp.float32),
            pltpu.VMEM((2, HSUB, D // 8, 1, 8, BSUB + 1), jnp.float32),
            pltpu.SemaphoreType.DMA((2,)),
            pltpu.SemaphoreType.DMA((2,)),
            pltpu.SemaphoreType.DMA((2,)),
        ],
        compiler_params=pltpu.CompilerParams(
            use_tc_tiling_on_sc=False, needs_layout_passes=False),
    )
    def k(table_hbm, idx_hbm, out_hbm, idx_v, gl_v, rows_v, tr_v,
          sem_i, sem_g, sem_o):
        wid = lax.axis_index("s") * NC + lax.axis_index("c")
        b_base = wid * b_per_w
        iota = lax.broadcasted_iota(jnp.int32, (L,), 0)

        def idx_copy(s, buf):
            h0 = (s % n_hsub) * HSUB
            b0 = b_base + (s // n_hsub) * BSUB
            return pltpu.make_async_copy(
                idx_hbm.at[pl.ds(h0, HSUB), pl.ds(b0, BSUB)],
                idx_v.at[buf], sem_i.at[buf])

        def out_copy(s, buf):
            h0 = (s % n_hsub) * HSUB
            b0 = b_base + (s // n_hsub) * BSUB
            return pltpu.make_async_copy(
                tr_v.at[buf, :, :, :, :, pl.ds(0, BSUB)],
                out_hbm.at[pl.ds(h0, HSUB), :, pl.ds(b0 // 128, 1), :,
                           pl.ds(b0 % 128, BSUB)],
                sem_o.at[buf])

        n_sub = n_bsub * n_hsub

        for buf in range(2):
            idx_copy(buf, buf).start()

        def step2(i2, carry):
            for buf in range(2):
                s = i2 * 2 + buf
                idx_copy(s, buf).wait()
                for hh in range(HSUB):
                    for half in range(BSUB // L):
                        v = idx_v[buf, hh, pl.ds(half * L, L)]
                        q = hh * (BSUB // L) + half
                        gl_v[buf, q // 8, pl.ds((q % 8) * L, L)] = v
                vpad = idx_v[buf, 0, pl.ds(0, L)]
                for q in range(ROWS // L, NG * GBLK // L):
                    gl_v[buf, q // 8, pl.ds((q % 8) * L, L)] = vpad
                gathers = [
                    pltpu.async_copy(
                        table_hbm.at[gl_v.at[buf, g]],
                        rows_v.at[buf, pl.ds(g * GBLK, GBLK), :],
                        sem_g.at[buf])
                    for g in range(NG)
                ]
                for gcp in gathers:
                    gcp.wait()
                @pl.when(s >= 2)
                def _():
                    out_copy(s - 2, buf).wait()

                zeros16 = jnp.zeros((L,), jnp.int32)
                i8a = iota // 8
                i8b = i8a + 2
                m8 = lax.rem(iota, jnp.full((L,), 8, jnp.int32))

                @plsc.parallel_loop(0, ROWS, unroll=8)
                def _(r):
                    hh = r // BSUB
                    bb = r % BSUB
                    bsplat = jnp.full((L,), bb, jnp.int32)
                    v0 = rows_v[buf, r, pl.ds(0, L)]
                    v1 = rows_v[buf, r, pl.ds(L, L)]
                    sub = tr_v.at[buf, hh]
                    plsc.store_scatter(sub, [i8a, zeros16, m8, bsplat], v0)
                    plsc.store_scatter(sub, [i8b, zeros16, m8, bsplat], v1)
                out_copy(s, buf).start()
                @pl.when(s + 2 < n_sub)
                def _():
                    idx_copy(s + 2, buf).start()
            return carry

        lax.fori_loop(0, n_sub // 2, step2, 0)
        for buf in range(2):
            out_copy(n_sub - 2 + buf, buf).wait()

    return k(table, idxT)


def kernel(idx, W_train, W_frozen):
    table = jnp.concatenate([W_train, W_frozen], axis=0)
    out5 = _sc_gather_t(table, idx.T)
    out = jnp.transpose(out5, (2, 4, 0, 1, 3))
    return out.reshape(B, H, D)
